# baseline (speedup 1.0000x reference)
import functools

import jax
import jax.numpy as jnp
from jax import lax
from jax.experimental import pallas as pl
from jax.experimental.pallas import tpu as pltpu
from jax.experimental.pallas import tpu_sc as plsc

N = 10000
E = 320000
D = 128
H = 128
L = 3

NC = 2
NS = 16
NW = NC * NS

EW = E // NW
CH = 80
CPW = 126
EWP = CPW * CH
TRASH = N
RPS = 632
RPS_LAST = N + 8 - (NS - 1) * RPS


@functools.lru_cache(maxsize=None)
def _build_sc_aggregate():
    mesh = plsc.VectorSubcoreMesh(core_axis_name="c", subcore_axis_name="s")

    @functools.partial(
        pl.kernel,
        out_type=jax.ShapeDtypeStruct((NC, N, D), jnp.float32),
        mesh=mesh,
        scratch_types=[
            pltpu.VMEM((CPW, CH), jnp.int32),
            pltpu.VMEM((CPW, CH), jnp.int32),
            pltpu.VMEM((CH, D), jnp.float32),
            pltpu.VMEM_SHARED((N + 8, D), jnp.float32),
            pltpu.SemaphoreType.DMA,
        ],
    )
    def _sc_aggregate(h_hbm, src_hbm, dst_hbm, zeros_hbm, out_hbm,
                      src_v, dst_v, rows_v, agg_sh, sem):
        cid = lax.axis_index("c")
        sid = lax.axis_index("s")
        wid = cid * NS + sid
        r0 = sid * RPS
        last = sid == NS - 1

        @pl.when(~last)
        def _():
            pltpu.sync_copy(zeros_hbm, agg_sh.at[pl.ds(r0, RPS)])

        @pl.when(last)
        def _():
            pltpu.sync_copy(zeros_hbm.at[pl.ds(0, RPS_LAST)],
                            agg_sh.at[pl.ds(r0, RPS_LAST)])

        pltpu.sync_copy(src_hbm.at[wid], src_v)
        pltpu.sync_copy(dst_hbm.at[wid], dst_v)

        plsc.subcore_barrier()

        def body(c, carry):
            pltpu.async_copy(h_hbm.at[src_v.at[c]], rows_v, sem).wait()
            pltpu.sync_copy(rows_v, agg_sh.at[dst_v.at[c]], add=True)
            return carry

        lax.fori_loop(0, CPW, body, 0)

        plsc.subcore_barrier()

        @pl.when(~last)
        def _():
            pltpu.sync_copy(agg_sh.at[pl.ds(r0, RPS)],
                            out_hbm.at[cid].at[pl.ds(r0, RPS)])

        @pl.when(last)
        def _():
            pltpu.sync_copy(agg_sh.at[pl.ds(r0, RPS_LAST - 8)],
                            out_hbm.at[cid].at[pl.ds(r0, RPS_LAST - 8)])

    return _sc_aggregate


BN = 1000
NBLK = N // BN


def _mlp_body(h_ref, r0_ref, r1_ref, w1_ref, b1_ref, w2_ref, b2_ref,
              rop_ref, wr_ref, br_ref, h_out_ref, ro_ref, y_ref):
    rst = h_ref[...] + r0_ref[...] + r1_ref[...]
    t = jnp.maximum(
        jnp.dot(rst, w1_ref[...], preferred_element_type=jnp.float32)
        + b1_ref[...][None, :], 0.0)
    o = (jnp.dot(t, w2_ref[...], preferred_element_type=jnp.float32)
         + b2_ref[...][None, :])
    h_out_ref[...] = o

    i = pl.program_id(0)

    @pl.when(i == 0)
    def _():
        ro_ref[...] = jnp.zeros_like(ro_ref)

    ro_ref[...] += jnp.sum(o, axis=0, keepdims=True)

    @pl.when(i == NBLK - 1)
    def _():
        hg = jnp.concatenate([rop_ref[...], ro_ref[...]], axis=0)
        y_ref[...] = (jnp.sum(hg * wr_ref[...]) + br_ref[0])[None, None]


def _mlp_layer(h, r0, r1, W1, b1, W2, b2, ro_prev, Wr_r, br):
    return pl.pallas_call(
        _mlp_body,
        grid=(NBLK,),
        in_specs=[
            pl.BlockSpec((BN, D), lambda i: (i, 0)),
            pl.BlockSpec((BN, D), lambda i: (i, 0)),
            pl.BlockSpec((BN, D), lambda i: (i, 0)),
            pl.BlockSpec((D, H), lambda i: (0, 0)),
            pl.BlockSpec((H,), lambda i: (0,)),
            pl.BlockSpec((H, H), lambda i: (0, 0)),
            pl.BlockSpec((H,), lambda i: (0,)),
            pl.BlockSpec((L - 1, H), lambda i: (0, 0)),
            pl.BlockSpec((L, H), lambda i: (0, 0)),
            pl.BlockSpec(memory_space=pltpu.SMEM),
        ],
        out_specs=[
            pl.BlockSpec((BN, H), lambda i: (i, 0)),
            pl.BlockSpec((1, H), lambda i: (0, 0)),
            pl.BlockSpec((1, 1), lambda i: (0, 0)),
        ],
        out_shape=[
            jax.ShapeDtypeStruct((N, H), jnp.float32),
            jax.ShapeDtypeStruct((1, H), jnp.float32),
            jax.ShapeDtypeStruct((1, 1), jnp.float32),
        ],
    )(h, r0, r1, W1, b1, W2, b2, ro_prev, Wr_r, br)


def kernel(h, edge_index, W1_0, b1_0, W2_0, b2_0, W1_1, b1_1, W2_1, b2_1,
           W1_2, b1_2, W2_2, b2_2, Wr, br):
    src = jnp.pad(edge_index[0].reshape(NW, EW), ((0, 0), (0, EWP - EW)),
                  constant_values=0).reshape(NW, CPW, CH)
    trash = (TRASH + (jnp.arange(NW, dtype=jnp.int32) % 8))[:, None]
    dst = jnp.concatenate(
        [edge_index[1].reshape(NW, EW),
         jnp.broadcast_to(trash, (NW, EWP - EW))], axis=1,
    ).reshape(NW, CPW, CH)
    zeros_init = jnp.zeros((RPS, D), jnp.float32)
    params = [(W1_0, b1_0, W2_0, b2_0), (W1_1, b1_1, W2_1, b2_1),
              (W1_2, b1_2, W2_2, b2_2)]
    Wr_r = Wr.reshape(L, H)

    sc_aggregate = _build_sc_aggregate()
    ro_prev = jnp.zeros((L - 1, H), jnp.float32)
    y = None
    for (W1, b1, W2, b2) in params:
        parts = sc_aggregate(h, src, dst, zeros_init)
        h, ro, y = _mlp_layer(h, parts[0], parts[1], W1, b1, W2, b2,
                              ro_prev, Wr_r, br)
        ro_prev = jnp.concatenate([ro_prev[1:], ro], axis=0)

    return y


# speedup vs baseline: 1.0018x; 1.0018x over previous
import functools

import jax
import jax.numpy as jnp
from jax import lax
from jax.experimental import pallas as pl
from jax.experimental.pallas import tpu as pltpu
from jax.experimental.pallas import tpu_sc as plsc

N = 10000
E = 320000
D = 128
H = 128
L = 3

NC = 2
NS = 16
NW = NC * NS

EW = E // NW
CH = 80
CPW = 126
EWP = CPW * CH
TRASH = N
RPS = 632
RPS_LAST = N + 8 - (NS - 1) * RPS


@functools.lru_cache(maxsize=None)
def _build_sc_aggregate():
    mesh = plsc.VectorSubcoreMesh(core_axis_name="c", subcore_axis_name="s")

    @functools.partial(
        pl.kernel,
        out_type=jax.ShapeDtypeStruct((NC, N, D), jnp.float32),
        mesh=mesh,
        scratch_types=[
            pltpu.VMEM((CPW, CH), jnp.int32),
            pltpu.VMEM((CPW, CH), jnp.int32),
            pltpu.VMEM((CH, D), jnp.float32),
            pltpu.VMEM_SHARED((N + 8, D), jnp.float32),
            pltpu.SemaphoreType.DMA,
        ],
    )
    def _sc_aggregate(h_hbm, src_hbm, dst_hbm, zeros_hbm, out_hbm,
                      src_v, dst_v, rows_v, agg_sh, sem):
        cid = lax.axis_index("c")
        sid = lax.axis_index("s")
        wid = cid * NS + sid
        r0 = sid * RPS
        last = sid == NS - 1

        @pl.when((cid == 0) & ~last)
        def _():
            pltpu.sync_copy(h_hbm.at[pl.ds(r0, RPS)],
                            agg_sh.at[pl.ds(r0, RPS)])

        @pl.when((cid == 0) & last)
        def _():
            pltpu.sync_copy(h_hbm.at[pl.ds(r0, RPS_LAST - 8)],
                            agg_sh.at[pl.ds(r0, RPS_LAST - 8)])

        @pl.when((cid == 0) & last)
        def _():
            pltpu.sync_copy(zeros_hbm.at[pl.ds(0, 8)],
                            agg_sh.at[pl.ds(N, 8)])

        @pl.when((cid != 0) & ~last)
        def _():
            pltpu.sync_copy(zeros_hbm, agg_sh.at[pl.ds(r0, RPS)])

        @pl.when((cid != 0) & last)
        def _():
            pltpu.sync_copy(zeros_hbm.at[pl.ds(0, RPS_LAST)],
                            agg_sh.at[pl.ds(r0, RPS_LAST)])

        pltpu.sync_copy(src_hbm.at[wid], src_v)
        pltpu.sync_copy(dst_hbm.at[wid], dst_v)

        plsc.subcore_barrier()

        def body(c, carry):
            pltpu.async_copy(h_hbm.at[src_v.at[c]], rows_v, sem).wait()
            pltpu.sync_copy(rows_v, agg_sh.at[dst_v.at[c]], add=True)
            return carry

        lax.fori_loop(0, CPW, body, 0)

        plsc.subcore_barrier()

        @pl.when(~last)
        def _():
            pltpu.sync_copy(agg_sh.at[pl.ds(r0, RPS)],
                            out_hbm.at[cid].at[pl.ds(r0, RPS)])

        @pl.when(last)
        def _():
            pltpu.sync_copy(agg_sh.at[pl.ds(r0, RPS_LAST - 8)],
                            out_hbm.at[cid].at[pl.ds(r0, RPS_LAST - 8)])

    return _sc_aggregate


BN = 1000
NBLK = N // BN


def _mlp_body(r0_ref, r1_ref, w1_ref, b1_ref, w2_ref, b2_ref,
              rop_ref, wr_ref, br_ref, h_out_ref, ro_ref, y_ref):
    rst = r0_ref[...] + r1_ref[...]
    t = jnp.maximum(
        jnp.dot(rst, w1_ref[...], preferred_element_type=jnp.float32)
        + b1_ref[...][None, :], 0.0)
    o = (jnp.dot(t, w2_ref[...], preferred_element_type=jnp.float32)
         + b2_ref[...][None, :])
    h_out_ref[...] = o

    i = pl.program_id(0)

    @pl.when(i == 0)
    def _():
        ro_ref[...] = jnp.zeros_like(ro_ref)

    ro_ref[...] += jnp.sum(o, axis=0, keepdims=True)

    @pl.when(i == NBLK - 1)
    def _():
        hg = jnp.concatenate([rop_ref[...], ro_ref[...]], axis=0)
        y_ref[...] = (jnp.sum(hg * wr_ref[...]) + br_ref[0])[None, None]


def _mlp_layer(r0, r1, W1, b1, W2, b2, ro_prev, Wr_r, br):
    return pl.pallas_call(
        _mlp_body,
        grid=(NBLK,),
        in_specs=[
            pl.BlockSpec((BN, D), lambda i: (i, 0)),
            pl.BlockSpec((BN, D), lambda i: (i, 0)),
            pl.BlockSpec((D, H), lambda i: (0, 0)),
            pl.BlockSpec((H,), lambda i: (0,)),
            pl.BlockSpec((H, H), lambda i: (0, 0)),
            pl.BlockSpec((H,), lambda i: (0,)),
            pl.BlockSpec((L - 1, H), lambda i: (0, 0)),
            pl.BlockSpec((L, H), lambda i: (0, 0)),
            pl.BlockSpec(memory_space=pltpu.SMEM),
        ],
        out_specs=[
            pl.BlockSpec((BN, H), lambda i: (i, 0)),
            pl.BlockSpec((1, H), lambda i: (0, 0)),
            pl.BlockSpec((1, 1), lambda i: (0, 0)),
        ],
        out_shape=[
            jax.ShapeDtypeStruct((N, H), jnp.float32),
            jax.ShapeDtypeStruct((1, H), jnp.float32),
            jax.ShapeDtypeStruct((1, 1), jnp.float32),
        ],
    )(r0, r1, W1, b1, W2, b2, ro_prev, Wr_r, br)


def kernel(h, edge_index, W1_0, b1_0, W2_0, b2_0, W1_1, b1_1, W2_1, b2_1,
           W1_2, b1_2, W2_2, b2_2, Wr, br):
    src = jnp.pad(edge_index[0].reshape(NW, EW), ((0, 0), (0, EWP - EW)),
                  constant_values=0).reshape(NW, CPW, CH)
    trash = (TRASH + (jnp.arange(NW, dtype=jnp.int32) % 8))[:, None]
    dst = jnp.concatenate(
        [edge_index[1].reshape(NW, EW),
         jnp.broadcast_to(trash, (NW, EWP - EW))], axis=1,
    ).reshape(NW, CPW, CH)
    zeros_init = jnp.zeros((RPS, D), jnp.float32)
    params = [(W1_0, b1_0, W2_0, b2_0), (W1_1, b1_1, W2_1, b2_1),
              (W1_2, b1_2, W2_2, b2_2)]
    Wr_r = Wr.reshape(L, H)

    sc_aggregate = _build_sc_aggregate()
    ro_prev = jnp.zeros((L - 1, H), jnp.float32)
    y = None
    for (W1, b1, W2, b2) in params:
        parts = sc_aggregate(h, src, dst, zeros_init)
        h, ro, y = _mlp_layer(parts[0], parts[1], W1, b1, W2, b2,
                              ro_prev, Wr_r, br)
        ro_prev = jnp.concatenate([ro_prev[1:], ro], axis=0)

    return y
